# Initial kernel scaffold; baseline (speedup 1.0000x reference)
import jax
import jax.numpy as jnp
from jax.experimental import pallas as pl


def kernel(x, edge_index, W_enc0, W_mu, W_logstd, W_l0, W_l1, W_l2):
    raise NotImplementedError("write your pallas kernel here")


# speedup vs baseline: 6.7727x; 6.7727x over previous
import functools

import jax
import jax.numpy as jnp
from jax import lax
from jax.experimental import pallas as pl
from jax.experimental.pallas import tpu as pltpu
from jax.experimental.pallas import tpu_sc as plsc

N = 4096
E = 131072
D = 512
H1 = 256
H2 = 128
H3 = 256
AR = 0.5

NC = 2
NS = 16
NW = NC * NS
EPW = E // NW
CHUNK = 128
RPT = N // NS

DW = 128
BM = 512
GRID = N // BM

PREC = lax.Precision.HIGHEST


def _dot(a, b, prec=PREC):
    return lax.dot_general(a, b, (((1,), (0,)), ((), ())), precision=prec,
                           preferred_element_type=jnp.float32)


def _dot_t(a, b, prec=PREC):
    return lax.dot_general(a, b, (((1,), (1,)), ((), ())), precision=prec,
                           preferred_element_type=jnp.float32)


SCW = 128


def _sc_gather_scatter(table, src, dst, an, gather=True):
    ne = src.shape[0]
    epw = ne // NW
    rpt = an // NS
    mesh = plsc.VectorSubcoreMesh(core_axis_name="c", subcore_axis_name="s")

    @functools.partial(
        pl.kernel,
        mesh=mesh,
        out_type=jax.ShapeDtypeStruct((NC, an, SCW), jnp.float32),
        scratch_types=[
            pltpu.VMEM((CHUNK,), jnp.int32),
            pltpu.VMEM((CHUNK,), jnp.int32),
            pltpu.VMEM((CHUNK, SCW), jnp.float32),
            pltpu.VMEM_SHARED((an, SCW), jnp.float32),
            pltpu.SemaphoreType.DMA,
        ],
    )
    def k(table_hbm, src_hbm, dst_hbm, zeros_hbm, out_hbm, srcv, dstv, rows,
          acc, sem):
        c = lax.axis_index("c")
        s = lax.axis_index("s")
        pltpu.sync_copy(zeros_hbm, acc.at[pl.ds(s * rpt, rpt)])
        if not gather:
            pltpu.sync_copy(table_hbm, rows)
        plsc.subcore_barrier()

        base = (s * NC + c) * epw

        def body(i, carry):
            off = pl.multiple_of(base + i * CHUNK, CHUNK)
            pltpu.sync_copy(dst_hbm.at[pl.ds(off, CHUNK)], dstv)
            if gather:
                pltpu.sync_copy(src_hbm.at[pl.ds(off, CHUNK)], srcv)
                pltpu.async_copy(table_hbm.at[srcv], rows, sem).wait()
            pltpu.sync_copy(rows, acc.at[dstv], add=True)
            return carry

        lax.fori_loop(0, epw // CHUNK, body, 0)
        plsc.subcore_barrier()
        pltpu.sync_copy(acc.at[pl.ds(s * rpt, rpt)],
                        out_hbm.at[c, pl.ds(s * rpt, rpt)])

    return k(table, src, dst, jnp.zeros((rpt, SCW), jnp.float32))


def _full(shape):
    nd = len(shape)
    return pl.BlockSpec(shape, lambda i, _nd=nd: (0,) * _nd)


def _rows(shape_blk, axis=0):
    def imap(i):
        idx = [0] * len(shape_blk)
        idx[axis] = i
        return tuple(idx)
    return pl.BlockSpec(shape_blk, imap)


def _dinv_from_partials(degp_blk):
    deg = jnp.sum(degp_blk, axis=(0, 2)) * (1.0 / DW) + 1.0
    return lax.rsqrt(deg)


def _mm_xw(x, wcat):
    def kfn(x_ref, w_ref, o_ref):
        o_ref[...] = _dot(x_ref[...], w_ref[...])

    return pl.pallas_call(
        kfn,
        grid=(GRID,),
        in_specs=[_rows((BM, D)), _full((D, H1 + H3))],
        out_specs=_rows((BM, H1 + H3)),
        out_shape=jax.ShapeDtypeStruct((N, H1 + H3), jnp.float32),
    )(x, wcat)


def _mm_scale_h0(degp, h0):
    def kfn(degp_ref, h0_ref, o_ref):
        dinv = _dinv_from_partials(degp_ref[...])
        o_ref[...] = h0_ref[...] * dinv[:, None]

    return pl.pallas_call(
        kfn,
        grid=(GRID,),
        in_specs=[_rows((NC, BM, DW), axis=1), _rows((BM, H1))],
        out_specs=_rows((BM, H1)),
        out_shape=jax.ShapeDtypeStruct((N, H1), jnp.float32),
    )(degp, h0)


def _mm_hidden(degp, p1, h0p, wmu):
    def kfn(degp_ref, p1_ref, h0p_ref, wmu_ref, hz_ref, hzp_ref):
        dinv = _dinv_from_partials(degp_ref[...])
        hidden1 = jax.nn.relu(
            (p1_ref[0] + p1_ref[1] + h0p_ref[...]) * dinv[:, None])
        hz = _dot(hidden1, wmu_ref[...])
        hz_ref[...] = hz
        hzp_ref[...] = hz * dinv[:, None]

    return pl.pallas_call(
        kfn,
        grid=(GRID,),
        in_specs=[_rows((NC, BM, DW), axis=1), _rows((NC, BM, H1), axis=1),
                  _rows((BM, H1)), _full((H1, H2))],
        out_specs=(_rows((BM, H2)), _rows((BM, H2))),
        out_shape=(jax.ShapeDtypeStruct((N, H2), jnp.float32),
                   jax.ShapeDtypeStruct((N, H2), jnp.float32)),
    )(degp, p1, h0p, wmu)


def _mm_z(degp, p2, hzp, wl1):
    def kfn(degp_ref, p2_ref, hzp_ref, wl1_ref, z_ref, v1_ref):
        dinv = _dinv_from_partials(degp_ref[...])
        z = (p2_ref[0] + p2_ref[1] + hzp_ref[...]) * dinv[:, None]
        z_ref[...] = z
        v1_ref[...] = _dot(z, wl1_ref[...])

    return pl.pallas_call(
        kfn,
        grid=(GRID,),
        in_specs=[_rows((NC, BM, DW), axis=1), _rows((NC, BM, H2), axis=1),
                  _rows((BM, H2)), _full((H2, H3))],
        out_specs=(_rows((BM, H2)), _rows((BM, H3))),
        out_shape=(jax.ShapeDtypeStruct((N, H2), jnp.float32),
                   jax.ShapeDtypeStruct((N, H3), jnp.float32)),
    )(degp, p2, hzp, wl1)


def _mm_sig(z, v1, v2):
    def kfn(zb_ref, zf_ref, v1_ref, v2_ref, s_ref, d_ref, dv1_ref, dv2_ref):
        logits = _dot_t(zb_ref[...], zf_ref[...])
        sig = jax.nn.sigmoid(logits)
        s_ref[...] = sig
        d = lax.rsqrt(jnp.sum(sig, axis=1))
        d_ref[...] = d.reshape(1, 1, BM)
        dv1_ref[...] = v1_ref[...] * d[:, None]
        dv2_ref[...] = v2_ref[...] * d[:, None]

    return pl.pallas_call(
        kfn,
        grid=(GRID,),
        in_specs=[_rows((BM, H2)), _full((N, H2)), _rows((BM, H3)),
                  _rows((BM, H3))],
        out_specs=(_rows((BM, N)), _rows((1, 1, BM)), _rows((BM, H3)),
                   _rows((BM, H3))),
        out_shape=(jax.ShapeDtypeStruct((N, N), jnp.float32),
                   jax.ShapeDtypeStruct((GRID, 1, BM), jnp.float32),
                   jax.ShapeDtypeStruct((N, H3), jnp.float32),
                   jax.ShapeDtypeStruct((N, H3), jnp.float32)),
    )(z, z, v1, v2)


def _mm_feedback(s, dv1, dv2, dvec, wl2):
    def kfn(s_ref, dv1_ref, dv2_ref, d_ref, wl2_ref, w2_ref):
        a1 = _dot(s_ref[...], dv1_ref[...])
        a2 = _dot(s_ref[...], dv2_ref[...])
        d = d_ref[0, 0, :]
        u = (jax.nn.relu(a1) + jax.nn.relu(a2)) * d[:, None]
        w2_ref[...] = _dot(u, wl2_ref[...]) * d[:, None]

    return pl.pallas_call(
        kfn,
        grid=(GRID,),
        in_specs=[_rows((BM, N)), _full((N, H3)), _full((N, H3)),
                  _rows((1, 1, BM)), _full((H3, H2))],
        out_specs=_rows((BM, H2)),
        out_shape=jax.ShapeDtypeStruct((N, H2), jnp.float32),
    )(s, dv1, dv2, dvec, wl2)


def _mm_update(s, w2, z, dvec):
    def kfn(s_ref, w2_ref, z_ref, d_ref, o_ref):
        d = d_ref[0, 0, :]
        upd = _dot(s_ref[...], w2_ref[...]) * d[:, None]
        o_ref[...] = (1.0 - AR) * z_ref[...] + AR * upd

    return pl.pallas_call(
        kfn,
        grid=(GRID,),
        in_specs=[_rows((BM, N)), _full((N, H2)), _rows((BM, H2)),
                  _rows((1, 1, BM))],
        out_specs=_rows((BM, H2)),
        out_shape=jax.ShapeDtypeStruct((N, H2), jnp.float32),
    )(s, w2, z, dvec)


def _mm_outer(upd):
    def kfn(ub_ref, uf_ref, o_ref):
        o_ref[...] = _dot_t(ub_ref[...], uf_ref[...])

    return pl.pallas_call(
        kfn,
        grid=(GRID,),
        in_specs=[_rows((BM, H2)), _full((N, H2))],
        out_specs=_rows((BM, N)),
        out_shape=jax.ShapeDtypeStruct((N, N), jnp.float32),
    )(upd, upd)


def kernel(x, edge_index, W_enc0, W_mu, W_logstd, W_l0, W_l1, W_l2):
    src = edge_index[0].astype(jnp.int32)
    dst = edge_index[1].astype(jnp.int32)

    ones_blk = jnp.ones((CHUNK, SCW), jnp.float32)
    degp = _sc_gather_scatter(ones_blk, dst, dst, N, gather=False)

    xw = _mm_xw(x, jnp.concatenate([W_enc0, W_l0], axis=1))
    h0 = xw[:, :H1]
    v2 = xw[:, H1:]

    h0p = _mm_scale_h0(degp, h0)
    h0p2 = jnp.concatenate([h0p[:, :SCW], h0p[:, SCW:]], axis=0)
    src2 = jnp.concatenate([src, src + N])
    dst2 = jnp.concatenate([dst, dst + N])
    p1d = _sc_gather_scatter(h0p2, src2, dst2, 2 * N)
    p1 = jnp.concatenate([p1d[:, :N, :], p1d[:, N:, :]], axis=2)
    hz, hzp = _mm_hidden(degp, p1, h0p, W_mu)
    p2 = _sc_gather_scatter(hzp, src, dst, N)
    z, v1 = _mm_z(degp, p2, hzp, W_l1)

    s, dvec, dv1, dv2 = _mm_sig(z, v1, v2)
    w2 = _mm_feedback(s, dv1, dv2, dvec, W_l2)
    upd = _mm_update(s, w2, z, dvec)
    out = _mm_outer(upd)
    return out.reshape(-1)


# speedup vs baseline: 13.7992x; 2.0375x over previous
import functools

import jax
import jax.numpy as jnp
from jax import lax
from jax.experimental import pallas as pl
from jax.experimental.pallas import tpu as pltpu
from jax.experimental.pallas import tpu_sc as plsc

N = 4096
E = 131072
D = 512
H1 = 256
H2 = 128
H3 = 256
AR = 0.5

NC = 2
NS = 16
NW = NC * NS
CHUNK = 128
SCW = 128

BM = 512
GRID = N // BM

PREC = lax.Precision.DEFAULT


def _dot(a, b, prec=PREC):
    return lax.dot_general(a, b, (((1,), (0,)), ((), ())), precision=prec,
                           preferred_element_type=jnp.float32)


def _dot_t(a, b, prec=PREC):
    return lax.dot_general(a, b, (((1,), (1,)), ((), ())), precision=prec,
                           preferred_element_type=jnp.float32)


def _sc_gather_scatter(table, src, dst, an, gather=True):
    ne = src.shape[0]
    epw = ne // NW
    nch = epw // CHUNK
    rpt = an // NS
    src2d = src.reshape(NW, nch, CHUNK)
    dst2d = dst.reshape(NW, nch, CHUNK)
    mesh = plsc.VectorSubcoreMesh(core_axis_name="c", subcore_axis_name="s")

    @functools.partial(
        pl.kernel,
        mesh=mesh,
        out_type=jax.ShapeDtypeStruct((NC, an, SCW), jnp.float32),
        scratch_types=[
            pltpu.VMEM((nch, CHUNK), jnp.int32),
            pltpu.VMEM((nch, CHUNK), jnp.int32),
            pltpu.VMEM((2, CHUNK, SCW), jnp.float32),
            pltpu.VMEM_SHARED((an, SCW), jnp.float32),
            pltpu.SemaphoreType.DMA,
            pltpu.SemaphoreType.DMA,
        ],
    )
    def k(table_hbm, src_hbm, dst_hbm, zeros_hbm, out_hbm, sidx, didx, rows,
          acc, sem0, sem1):
        c = lax.axis_index("c")
        s = lax.axis_index("s")
        wid = s * NC + c
        pltpu.sync_copy(zeros_hbm, acc.at[pl.ds(s * rpt, rpt)])
        pltpu.sync_copy(dst_hbm.at[wid], didx)
        if gather:
            pltpu.sync_copy(src_hbm.at[wid], sidx)
        else:
            pltpu.sync_copy(table_hbm, rows.at[0])
        plsc.subcore_barrier()

        if gather:
            sems = (sem0, sem1)

            def gdesc(i, b):
                return pltpu.make_async_copy(
                    table_hbm.at[sidx.at[i]], rows.at[b], sems[b])

            for b in range(2):
                gdesc(b, b).start()

            def body(j, carry):
                for b in range(2):
                    i = j * 2 + b
                    gdesc(i, b).wait()
                    pltpu.sync_copy(rows.at[b], acc.at[didx.at[i]], add=True)

                    @pl.when(i + 2 < nch)
                    def _():
                        gdesc(i + 2, b).start()
                return carry

            lax.fori_loop(0, nch // 2, body, 0)
        else:
            def sdesc(i):
                return pltpu.make_async_copy(
                    rows.at[0], acc.at[didx.at[i]], sem0)

            def fire(i, carry):
                sdesc(i).start(add=True)
                return carry

            def drain(i, carry):
                sdesc(i).wait()
                return carry

            lax.fori_loop(0, nch, fire, 0)
            lax.fori_loop(0, nch, drain, 0)

        plsc.subcore_barrier()
        pltpu.sync_copy(acc.at[pl.ds(s * rpt, rpt)],
                        out_hbm.at[c, pl.ds(s * rpt, rpt)])

    return k(table, src2d, dst2d, jnp.zeros((rpt, SCW), jnp.float32))


def _full(shape):
    nd = len(shape)
    return pl.BlockSpec(shape, lambda i, _nd=nd: (0,) * _nd)


def _rows(shape_blk, axis=0):
    def imap(i):
        idx = [0] * len(shape_blk)
        idx[axis] = i
        return tuple(idx)
    return pl.BlockSpec(shape_blk, imap)


def _dinv_from_partials(degp_blk):
    deg = jnp.sum(degp_blk, axis=(0, 2)) * (1.0 / SCW) + 1.0
    return lax.rsqrt(deg)


def _mm_xw(x, wcat):
    def kfn(x_ref, w_ref, h0st_ref, v2_ref):
        t = _dot(x_ref[...], w_ref[...])
        h0st_ref[0] = t[:, :SCW]
        h0st_ref[1] = t[:, SCW:H1]
        v2_ref[...] = t[:, H1:]

    return pl.pallas_call(
        kfn,
        grid=(GRID,),
        in_specs=[_rows((BM, D)), _full((D, H1 + H3))],
        out_specs=(_rows((2, BM, SCW), axis=1), _rows((BM, H3))),
        out_shape=(jax.ShapeDtypeStruct((2, N, SCW), jnp.float32),
                   jax.ShapeDtypeStruct((N, H3), jnp.float32)),
    )(x, wcat)


def _mm_scale_h0(degp, h0st):
    def kfn(degp_ref, h0_ref, o_ref):
        dinv = _dinv_from_partials(degp_ref[...])
        o_ref[...] = h0_ref[...] * dinv[None, :, None]

    return pl.pallas_call(
        kfn,
        grid=(GRID,),
        in_specs=[_rows((NC, BM, SCW), axis=1), _rows((2, BM, SCW), axis=1)],
        out_specs=_rows((2, BM, SCW), axis=1),
        out_shape=jax.ShapeDtypeStruct((2, N, SCW), jnp.float32),
    )(degp, h0st)


def _mm_hidden(degp, p1d, h0pst, wmu):
    def kfn(degp_ref, p1_ref, h0p_ref, wmu_ref, hz_ref, hzp_ref):
        dinv = _dinv_from_partials(degp_ref[...])
        left = p1_ref[0, 0] + p1_ref[1, 0] + h0p_ref[0]
        right = p1_ref[0, 1] + p1_ref[1, 1] + h0p_ref[1]
        hidden1 = jax.nn.relu(
            jnp.concatenate([left, right], axis=1) * dinv[:, None])
        hz = _dot(hidden1, wmu_ref[...])
        hz_ref[...] = hz
        hzp_ref[...] = hz * dinv[:, None]

    return pl.pallas_call(
        kfn,
        grid=(GRID,),
        in_specs=[_rows((NC, BM, SCW), axis=1),
                  _rows((NC, 2, BM, SCW), axis=2),
                  _rows((2, BM, SCW), axis=1), _full((H1, H2))],
        out_specs=(_rows((BM, H2)), _rows((BM, H2))),
        out_shape=(jax.ShapeDtypeStruct((N, H2), jnp.float32),
                   jax.ShapeDtypeStruct((N, H2), jnp.float32)),
    )(degp, p1d, h0pst, wmu)


def _mm_z(degp, p2, hzp, wl1):
    def kfn(degp_ref, p2_ref, hzp_ref, wl1_ref, z_ref, v1_ref):
        dinv = _dinv_from_partials(degp_ref[...])
        z = (p2_ref[0] + p2_ref[1] + hzp_ref[...]) * dinv[:, None]
        z_ref[...] = z
        v1_ref[...] = _dot(z, wl1_ref[...])

    return pl.pallas_call(
        kfn,
        grid=(GRID,),
        in_specs=[_rows((NC, BM, SCW), axis=1), _rows((NC, BM, H2), axis=1),
                  _rows((BM, H2)), _full((H2, H3))],
        out_specs=(_rows((BM, H2)), _rows((BM, H3))),
        out_shape=(jax.ShapeDtypeStruct((N, H2), jnp.float32),
                   jax.ShapeDtypeStruct((N, H3), jnp.float32)),
    )(degp, p2, hzp, wl1)


def _mm_sig(z, v1, v2):
    def kfn(zb_ref, zf_ref, v1_ref, v2_ref, s_ref, d_ref, dv1_ref, dv2_ref):
        logits = _dot_t(zb_ref[...], zf_ref[...])
        sig = jax.nn.sigmoid(logits)
        s_ref[...] = sig
        d = lax.rsqrt(jnp.sum(sig, axis=1))
        d_ref[...] = d.reshape(1, 1, BM)
        dv1_ref[...] = v1_ref[...] * d[:, None]
        dv2_ref[...] = v2_ref[...] * d[:, None]

    return pl.pallas_call(
        kfn,
        grid=(GRID,),
        in_specs=[_rows((BM, H2)), _full((N, H2)), _rows((BM, H3)),
                  _rows((BM, H3))],
        out_specs=(_rows((BM, N)), _rows((1, 1, BM)), _rows((BM, H3)),
                   _rows((BM, H3))),
        out_shape=(jax.ShapeDtypeStruct((N, N), jnp.float32),
                   jax.ShapeDtypeStruct((GRID, 1, BM), jnp.float32),
                   jax.ShapeDtypeStruct((N, H3), jnp.float32),
                   jax.ShapeDtypeStruct((N, H3), jnp.float32)),
    )(z, z, v1, v2)


def _mm_feedback(s, dv1, dv2, dvec, wl2):
    def kfn(s_ref, dv1_ref, dv2_ref, d_ref, wl2_ref, w2_ref):
        a1 = _dot(s_ref[...], dv1_ref[...])
        a2 = _dot(s_ref[...], dv2_ref[...])
        d = d_ref[0, 0, :]
        u = (jax.nn.relu(a1) + jax.nn.relu(a2)) * d[:, None]
        w2_ref[...] = _dot(u, wl2_ref[...]) * d[:, None]

    return pl.pallas_call(
        kfn,
        grid=(GRID,),
        in_specs=[_rows((BM, N)), _full((N, H3)), _full((N, H3)),
                  _rows((1, 1, BM)), _full((H3, H2))],
        out_specs=_rows((BM, H2)),
        out_shape=jax.ShapeDtypeStruct((N, H2), jnp.float32),
    )(s, dv1, dv2, dvec, wl2)


def _mm_update(s, w2, z, dvec):
    def kfn(s_ref, w2_ref, z_ref, d_ref, o_ref):
        d = d_ref[0, 0, :]
        upd = _dot(s_ref[...], w2_ref[...]) * d[:, None]
        o_ref[...] = (1.0 - AR) * z_ref[...] + AR * upd

    return pl.pallas_call(
        kfn,
        grid=(GRID,),
        in_specs=[_rows((BM, N)), _full((N, H2)), _rows((BM, H2)),
                  _rows((1, 1, BM))],
        out_specs=_rows((BM, H2)),
        out_shape=jax.ShapeDtypeStruct((N, H2), jnp.float32),
    )(s, w2, z, dvec)


def _mm_outer(upd):
    def kfn(ub_ref, uf_ref, o_ref):
        o_ref[...] = _dot_t(ub_ref[...], uf_ref[...])

    return pl.pallas_call(
        kfn,
        grid=(GRID,),
        in_specs=[_rows((BM, H2)), _full((N, H2))],
        out_specs=_rows((BM, N)),
        out_shape=jax.ShapeDtypeStruct((N, N), jnp.float32),
    )(upd, upd)


def kernel(x, edge_index, W_enc0, W_mu, W_logstd, W_l0, W_l1, W_l2):
    src = edge_index[0].astype(jnp.int32)
    dst = edge_index[1].astype(jnp.int32)

    ones_blk = jnp.ones((CHUNK, SCW), jnp.float32)
    degp = _sc_gather_scatter(ones_blk, dst, dst, N, gather=False)

    h0st, v2 = _mm_xw(x, jnp.concatenate([W_enc0, W_l0], axis=1))
    h0pst = _mm_scale_h0(degp, h0st)

    src2 = jnp.concatenate([src, src + N])
    dst2 = jnp.concatenate([dst, dst + N])
    p1d = _sc_gather_scatter(h0pst.reshape(2 * N, SCW), src2, dst2, 2 * N)
    hz, hzp = _mm_hidden(degp, p1d.reshape(NC, 2, N, SCW), h0pst, W_mu)
    p2 = _sc_gather_scatter(hzp, src, dst, N)
    z, v1 = _mm_z(degp, p2, hzp, W_l1)

    s, dvec, dv1, dv2 = _mm_sig(z, v1, v2)
    w2 = _mm_feedback(s, dv1, dv2, dvec, W_l2)
    upd = _mm_update(s, w2, z, dvec)
    out = _mm_outer(upd)
    return out.reshape(-1)


# speedup vs baseline: 15.6496x; 1.1341x over previous
import functools

import jax
import jax.numpy as jnp
from jax import lax
from jax.experimental import pallas as pl
from jax.experimental.pallas import tpu as pltpu
from jax.experimental.pallas import tpu_sc as plsc

N = 4096
E = 131072
D = 512
H1 = 256
H2 = 128
H3 = 256
AR = 0.5

NC = 2
NS = 16
NW = NC * NS
CHUNK = 128
SCW = 128

BM = 512
GRID = N // BM

PREC = lax.Precision.DEFAULT


def _dot(a, b, prec=PREC):
    return lax.dot_general(a, b, (((1,), (0,)), ((), ())), precision=prec,
                           preferred_element_type=jnp.float32)


def _dot_t(a, b, prec=PREC):
    return lax.dot_general(a, b, (((1,), (1,)), ((), ())), precision=prec,
                           preferred_element_type=jnp.float32)


def _sc_gather_scatter(table, src, dst, an, gather=True):
    ne = src.shape[0]
    epw = ne // NW
    nch = epw // CHUNK
    rpt = an // NS
    src2d = src.reshape(NW, nch, CHUNK)
    dst2d = dst.reshape(NW, nch, CHUNK)
    mesh = plsc.VectorSubcoreMesh(core_axis_name="c", subcore_axis_name="s")

    @functools.partial(
        pl.kernel,
        mesh=mesh,
        out_type=jax.ShapeDtypeStruct((NC, an, SCW), jnp.float32),
        scratch_types=[
            pltpu.VMEM((nch, CHUNK), jnp.int32),
            pltpu.VMEM((nch, CHUNK), jnp.int32),
            pltpu.VMEM((2, CHUNK, SCW), jnp.float32),
            pltpu.VMEM_SHARED((an, SCW), jnp.float32),
            pltpu.SemaphoreType.DMA,
            pltpu.SemaphoreType.DMA,
        ],
    )
    def k(table_hbm, src_hbm, dst_hbm, zeros_hbm, out_hbm, sidx, didx, rows,
          acc, sem0, sem1):
        c = lax.axis_index("c")
        s = lax.axis_index("s")
        wid = s * NC + c
        pltpu.sync_copy(zeros_hbm, acc.at[pl.ds(s * rpt, rpt)])
        pltpu.sync_copy(dst_hbm.at[wid], didx)
        if gather:
            pltpu.sync_copy(src_hbm.at[wid], sidx)
        else:
            pltpu.sync_copy(table_hbm, rows.at[0])
        plsc.subcore_barrier()

        if gather:
            sems = (sem0, sem1)

            def gdesc(i, b):
                return pltpu.make_async_copy(
                    table_hbm.at[sidx.at[i]], rows.at[b], sems[b])

            for b in range(2):
                gdesc(b, b).start()

            def body(j, carry):
                for b in range(2):
                    i = j * 2 + b
                    gdesc(i, b).wait()
                    pltpu.sync_copy(rows.at[b], acc.at[didx.at[i]], add=True)

                    @pl.when(i + 2 < nch)
                    def _():
                        gdesc(i + 2, b).start()
                return carry

            lax.fori_loop(0, nch // 2, body, 0)
        else:
            def sdesc(i):
                return pltpu.make_async_copy(
                    rows.at[0], acc.at[didx.at[i]], sem0)

            def fire(i, carry):
                sdesc(i).start(add=True)
                return carry

            def drain(i, carry):
                sdesc(i).wait()
                return carry

            lax.fori_loop(0, nch, fire, 0)
            lax.fori_loop(0, nch, drain, 0)

        plsc.subcore_barrier()
        pltpu.sync_copy(acc.at[pl.ds(s * rpt, rpt)],
                        out_hbm.at[c, pl.ds(s * rpt, rpt)])

    return k(table, src2d, dst2d, jnp.zeros((rpt, SCW), jnp.float32))


def _full(shape):
    nd = len(shape)
    return pl.BlockSpec(shape, lambda i, _nd=nd: (0,) * _nd)


def _rows(shape_blk, axis=0):
    def imap(i):
        idx = [0] * len(shape_blk)
        idx[axis] = i
        return tuple(idx)
    return pl.BlockSpec(shape_blk, imap)


def _dinv_from_partials(degp_blk):
    deg = jnp.sum(degp_blk, axis=(0, 2)) * (1.0 / SCW) + 1.0
    return lax.rsqrt(deg)


def _mm_xw(x, wcat):
    def kfn(x_ref, w_ref, h0st_ref, v2_ref):
        t = _dot(x_ref[...], w_ref[...])
        h0st_ref[0] = t[:, :SCW]
        h0st_ref[1] = t[:, SCW:H1]
        v2_ref[...] = t[:, H1:]

    return pl.pallas_call(
        kfn,
        grid=(GRID,),
        in_specs=[_rows((BM, D)), _full((D, H1 + H3))],
        out_specs=(_rows((2, BM, SCW), axis=1), _rows((BM, H3))),
        out_shape=(jax.ShapeDtypeStruct((2, N, SCW), jnp.float32),
                   jax.ShapeDtypeStruct((N, H3), jnp.float32)),
    )(x, wcat)


def _mm_scale_h0(degp, h0st):
    def kfn(degp_ref, h0_ref, o_ref):
        dinv = _dinv_from_partials(degp_ref[...])
        o_ref[...] = h0_ref[...] * dinv[None, :, None]

    return pl.pallas_call(
        kfn,
        grid=(GRID,),
        in_specs=[_rows((NC, BM, SCW), axis=1), _rows((2, BM, SCW), axis=1)],
        out_specs=_rows((2, BM, SCW), axis=1),
        out_shape=jax.ShapeDtypeStruct((2, N, SCW), jnp.float32),
    )(degp, h0st)


def _mm_hidden(degp, p1d, h0pst, wmu):
    def kfn(degp_ref, p1_ref, h0p_ref, wmu_ref, hz_ref, hzp_ref):
        dinv = _dinv_from_partials(degp_ref[...])
        left = p1_ref[0, 0] + p1_ref[1, 0] + h0p_ref[0]
        right = p1_ref[0, 1] + p1_ref[1, 1] + h0p_ref[1]
        hidden1 = jax.nn.relu(
            jnp.concatenate([left, right], axis=1) * dinv[:, None])
        hz = _dot(hidden1, wmu_ref[...])
        hz_ref[...] = hz
        hzp_ref[...] = hz * dinv[:, None]

    return pl.pallas_call(
        kfn,
        grid=(GRID,),
        in_specs=[_rows((NC, BM, SCW), axis=1),
                  _rows((NC, 2, BM, SCW), axis=2),
                  _rows((2, BM, SCW), axis=1), _full((H1, H2))],
        out_specs=(_rows((BM, H2)), _rows((BM, H2))),
        out_shape=(jax.ShapeDtypeStruct((N, H2), jnp.float32),
                   jax.ShapeDtypeStruct((N, H2), jnp.float32)),
    )(degp, p1d, h0pst, wmu)


def _mm_z(degp, p2, hzp, wl1):
    def kfn(degp_ref, p2_ref, hzp_ref, wl1_ref, z_ref, v1_ref):
        dinv = _dinv_from_partials(degp_ref[...])
        z = (p2_ref[0] + p2_ref[1] + hzp_ref[...]) * dinv[:, None]
        z_ref[...] = z
        v1_ref[...] = _dot(z, wl1_ref[...])

    return pl.pallas_call(
        kfn,
        grid=(GRID,),
        in_specs=[_rows((NC, BM, SCW), axis=1), _rows((NC, BM, H2), axis=1),
                  _rows((BM, H2)), _full((H2, H3))],
        out_specs=(_rows((BM, H2)), _rows((BM, H3))),
        out_shape=(jax.ShapeDtypeStruct((N, H2), jnp.float32),
                   jax.ShapeDtypeStruct((N, H3), jnp.float32)),
    )(degp, p2, hzp, wl1)


def _mm_sig(z, v1, v2):
    def kfn(zb_ref, zf_ref, v1_ref, v2_ref, s_ref, d_ref, dv1_ref, dv2_ref):
        logits = _dot_t(zb_ref[...], zf_ref[...])
        sig = jax.nn.sigmoid(logits)
        s_ref[...] = sig
        d = lax.rsqrt(jnp.sum(sig, axis=1))
        d_ref[...] = d.reshape(1, 1, BM)
        dv1_ref[...] = v1_ref[...] * d[:, None]
        dv2_ref[...] = v2_ref[...] * d[:, None]

    return pl.pallas_call(
        kfn,
        grid=(GRID,),
        in_specs=[_rows((BM, H2)), _full((N, H2)), _rows((BM, H3)),
                  _rows((BM, H3))],
        out_specs=(_rows((BM, N)), _rows((1, 1, BM)), _rows((BM, H3)),
                   _rows((BM, H3))),
        out_shape=(jax.ShapeDtypeStruct((N, N), jnp.float32),
                   jax.ShapeDtypeStruct((GRID, 1, BM), jnp.float32),
                   jax.ShapeDtypeStruct((N, H3), jnp.float32),
                   jax.ShapeDtypeStruct((N, H3), jnp.float32)),
    )(z, z, v1, v2)


def _mm_feedback(s, dv1, dv2, dvec, wl2):
    def kfn(s_ref, dv1_ref, dv2_ref, d_ref, wl2_ref, w2_ref):
        a1 = _dot(s_ref[...], dv1_ref[...])
        a2 = _dot(s_ref[...], dv2_ref[...])
        d = d_ref[0, 0, :]
        u = (jax.nn.relu(a1) + jax.nn.relu(a2)) * d[:, None]
        w2_ref[...] = _dot(u, wl2_ref[...]) * d[:, None]

    return pl.pallas_call(
        kfn,
        grid=(GRID,),
        in_specs=[_rows((BM, N)), _full((N, H3)), _full((N, H3)),
                  _rows((1, 1, BM)), _full((H3, H2))],
        out_specs=_rows((BM, H2)),
        out_shape=jax.ShapeDtypeStruct((N, H2), jnp.float32),
    )(s, dv1, dv2, dvec, wl2)


def _mm_update(s, w2, z, dvec):
    def kfn(s_ref, w2_ref, z_ref, d_ref, o_ref):
        d = d_ref[0, 0, :]
        upd = _dot(s_ref[...], w2_ref[...]) * d[:, None]
        o_ref[...] = (1.0 - AR) * z_ref[...] + AR * upd

    return pl.pallas_call(
        kfn,
        grid=(GRID,),
        in_specs=[_rows((BM, N)), _full((N, H2)), _rows((BM, H2)),
                  _rows((1, 1, BM))],
        out_specs=_rows((BM, H2)),
        out_shape=jax.ShapeDtypeStruct((N, H2), jnp.float32),
    )(s, w2, z, dvec)


def _mm_outer(upd):
    def kfn(ub_ref, uf_ref, o_ref):
        t = _dot_t(ub_ref[...], uf_ref[...])
        o_ref[...] = t.reshape(BM, N // SCW, SCW)

    return pl.pallas_call(
        kfn,
        grid=(GRID,),
        in_specs=[_rows((BM, H2)), _full((N, H2))],
        out_specs=_rows((BM, N // SCW, SCW)),
        out_shape=jax.ShapeDtypeStruct((N, N // SCW, SCW), jnp.float32),
    )(upd, upd)


def kernel(x, edge_index, W_enc0, W_mu, W_logstd, W_l0, W_l1, W_l2):
    src = edge_index[0].astype(jnp.int32)
    dst = edge_index[1].astype(jnp.int32)

    ones_blk = jnp.ones((CHUNK, SCW), jnp.float32)
    degp = _sc_gather_scatter(ones_blk, dst, dst, N, gather=False)

    h0st, v2 = _mm_xw(x, jnp.concatenate([W_enc0, W_l0], axis=1))
    h0pst = _mm_scale_h0(degp, h0st)

    src2 = jnp.concatenate([src, src + N])
    dst2 = jnp.concatenate([dst, dst + N])
    p1d = _sc_gather_scatter(h0pst.reshape(2 * N, SCW), src2, dst2, 2 * N)
    hz, hzp = _mm_hidden(degp, p1d.reshape(NC, 2, N, SCW), h0pst, W_mu)
    p2 = _sc_gather_scatter(hzp, src, dst, N)
    z, v1 = _mm_z(degp, p2, hzp, W_l1)

    s, dvec, dv1, dv2 = _mm_sig(z, v1, v2)
    w2 = _mm_feedback(s, dv1, dv2, dvec, W_l2)
    upd = _mm_update(s, w2, z, dvec)
    out = _mm_outer(upd)
    return out.reshape(-1)


# speedup vs baseline: 16.4464x; 1.0509x over previous
import functools

import jax
import jax.numpy as jnp
from jax import lax
from jax.experimental import pallas as pl
from jax.experimental.pallas import tpu as pltpu
from jax.experimental.pallas import tpu_sc as plsc

N = 4096
E = 131072
D = 512
H1 = 256
H2 = 128
H3 = 256
AR = 0.5

NC = 2
NS = 16
NW = NC * NS
CHUNK = 128
SCW = 128

BM = 512
GRID = N // BM

PREC = lax.Precision.DEFAULT


def _dot(a, b, prec=PREC):
    return lax.dot_general(a, b, (((1,), (0,)), ((), ())), precision=prec,
                           preferred_element_type=jnp.float32)


def _dot_t(a, b, prec=PREC):
    return lax.dot_general(a, b, (((1,), (1,)), ((), ())), precision=prec,
                           preferred_element_type=jnp.float32)


def _sc_gather_scatter(table, src, dst, an, gather=True):
    ne = src.shape[0]
    epw = ne // NW
    nch = epw // CHUNK
    rpt = an // NS
    src2d = src.reshape(NW, nch, CHUNK)
    dst2d = dst.reshape(NW, nch, CHUNK)
    mesh = plsc.VectorSubcoreMesh(core_axis_name="c", subcore_axis_name="s")

    @functools.partial(
        pl.kernel,
        mesh=mesh,
        out_type=jax.ShapeDtypeStruct((NC, an, SCW), jnp.float32),
        scratch_types=[
            pltpu.VMEM((nch, CHUNK), jnp.int32),
            pltpu.VMEM((nch, CHUNK), jnp.int32),
            pltpu.VMEM((2, CHUNK, SCW), jnp.float32),
            pltpu.VMEM_SHARED((an, SCW), jnp.float32),
            pltpu.SemaphoreType.DMA,
            pltpu.SemaphoreType.DMA,
        ],
    )
    def k(table_hbm, src_hbm, dst_hbm, zeros_hbm, out_hbm, sidx, didx, rows,
          acc, sem0, sem1):
        c = lax.axis_index("c")
        s = lax.axis_index("s")
        wid = s * NC + c
        pltpu.sync_copy(zeros_hbm, acc.at[pl.ds(s * rpt, rpt)])
        pltpu.sync_copy(dst_hbm.at[wid], didx)
        if gather:
            pltpu.sync_copy(src_hbm.at[wid], sidx)
        else:
            pltpu.sync_copy(table_hbm, rows.at[0])
        plsc.subcore_barrier()

        if gather:
            sems = (sem0, sem1)

            def gdesc(i, b):
                return pltpu.make_async_copy(
                    table_hbm.at[sidx.at[i]], rows.at[b], sems[b])

            for b in range(2):
                gdesc(b, b).start()

            def body(j, carry):
                for b in range(2):
                    i = j * 2 + b
                    gdesc(i, b).wait()
                    pltpu.sync_copy(rows.at[b], acc.at[didx.at[i]], add=True)

                    @pl.when(i + 2 < nch)
                    def _():
                        gdesc(i + 2, b).start()
                return carry

            lax.fori_loop(0, nch // 2, body, 0)
        else:
            def sdesc(i):
                return pltpu.make_async_copy(
                    rows.at[0], acc.at[didx.at[i]], sem0)

            def fire(i, carry):
                sdesc(i).start(add=True)
                return carry

            def drain(i, carry):
                sdesc(i).wait()
                return carry

            lax.fori_loop(0, nch, fire, 0)
            lax.fori_loop(0, nch, drain, 0)

        plsc.subcore_barrier()
        pltpu.sync_copy(acc.at[pl.ds(s * rpt, rpt)],
                        out_hbm.at[c, pl.ds(s * rpt, rpt)])

    return k(table, src2d, dst2d, jnp.zeros((rpt, SCW), jnp.float32))


def _full(shape):
    nd = len(shape)
    return pl.BlockSpec(shape, lambda i, _nd=nd: (0,) * _nd)


def _rows(shape_blk, axis=0):
    def imap(i):
        idx = [0] * len(shape_blk)
        idx[axis] = i
        return tuple(idx)
    return pl.BlockSpec(shape_blk, imap)


def _dinv_from_partials(degp_blk):
    deg = jnp.sum(degp_blk, axis=(0, 2)) * (1.0 / SCW) + 1.0
    return lax.rsqrt(deg)


def _mm_xw(x, wcat):
    def kfn(x_ref, w_ref, h0st_ref, v2_ref):
        t = _dot(x_ref[...], w_ref[...])
        h0st_ref[0] = t[:, :SCW]
        h0st_ref[1] = t[:, SCW:H1]
        v2_ref[...] = t[:, H1:]

    return pl.pallas_call(
        kfn,
        grid=(GRID,),
        in_specs=[_rows((BM, D)), _full((D, H1 + H3))],
        out_specs=(_rows((2, BM, SCW), axis=1), _rows((BM, H3))),
        out_shape=(jax.ShapeDtypeStruct((2, N, SCW), jnp.float32),
                   jax.ShapeDtypeStruct((N, H3), jnp.float32)),
    )(x, wcat)


def _mm_scale_h0(degp, h0st):
    def kfn(degp_ref, h0_ref, o_ref):
        dinv = _dinv_from_partials(degp_ref[...])
        o_ref[...] = h0_ref[...] * dinv[None, :, None]

    return pl.pallas_call(
        kfn,
        grid=(GRID,),
        in_specs=[_rows((NC, BM, SCW), axis=1), _rows((2, BM, SCW), axis=1)],
        out_specs=_rows((2, BM, SCW), axis=1),
        out_shape=jax.ShapeDtypeStruct((2, N, SCW), jnp.float32),
    )(degp, h0st)


def _mm_hidden(degp, p1d, h0pst, wmu):
    def kfn(degp_ref, p1_ref, h0p_ref, wmu_ref, hz_ref, hzp_ref):
        dinv = _dinv_from_partials(degp_ref[...])
        left = p1_ref[0, 0] + p1_ref[1, 0] + h0p_ref[0]
        right = p1_ref[0, 1] + p1_ref[1, 1] + h0p_ref[1]
        hidden1 = jax.nn.relu(
            jnp.concatenate([left, right], axis=1) * dinv[:, None])
        hz = _dot(hidden1, wmu_ref[...])
        hz_ref[...] = hz
        hzp_ref[...] = hz * dinv[:, None]

    return pl.pallas_call(
        kfn,
        grid=(GRID,),
        in_specs=[_rows((NC, BM, SCW), axis=1),
                  _rows((NC, 2, BM, SCW), axis=2),
                  _rows((2, BM, SCW), axis=1), _full((H1, H2))],
        out_specs=(_rows((BM, H2)), _rows((BM, H2))),
        out_shape=(jax.ShapeDtypeStruct((N, H2), jnp.float32),
                   jax.ShapeDtypeStruct((N, H2), jnp.float32)),
    )(degp, p1d, h0pst, wmu)


def _mm_z(degp, p2, hzp, wl1):
    def kfn(degp_ref, p2_ref, hzp_ref, wl1_ref, z_ref, v1_ref):
        dinv = _dinv_from_partials(degp_ref[...])
        z = (p2_ref[0] + p2_ref[1] + hzp_ref[...]) * dinv[:, None]
        z_ref[...] = z
        v1_ref[...] = _dot(z, wl1_ref[...])

    return pl.pallas_call(
        kfn,
        grid=(GRID,),
        in_specs=[_rows((NC, BM, SCW), axis=1), _rows((NC, BM, H2), axis=1),
                  _rows((BM, H2)), _full((H2, H3))],
        out_specs=(_rows((BM, H2)), _rows((BM, H3))),
        out_shape=(jax.ShapeDtypeStruct((N, H2), jnp.float32),
                   jax.ShapeDtypeStruct((N, H3), jnp.float32)),
    )(degp, p2, hzp, wl1)


def _mm_sig(z, v1, v2):
    def kfn(zb_ref, zf_ref, v1_ref, v2_ref, s_ref, d_ref, dv1_ref, dv2_ref):
        logits = _dot_t(zb_ref[...], zf_ref[...])
        sig = jax.nn.sigmoid(logits)
        s_ref[...] = sig.astype(jnp.bfloat16)
        d = lax.rsqrt(jnp.sum(sig, axis=1))
        d_ref[...] = d.reshape(1, 1, BM)
        dv1_ref[...] = (v1_ref[...] * d[:, None]).astype(jnp.bfloat16)
        dv2_ref[...] = (v2_ref[...] * d[:, None]).astype(jnp.bfloat16)

    return pl.pallas_call(
        kfn,
        grid=(GRID,),
        in_specs=[_rows((BM, H2)), _full((N, H2)), _rows((BM, H3)),
                  _rows((BM, H3))],
        out_specs=(_rows((BM, N)), _rows((1, 1, BM)), _rows((BM, H3)),
                   _rows((BM, H3))),
        out_shape=(jax.ShapeDtypeStruct((N, N), jnp.bfloat16),
                   jax.ShapeDtypeStruct((GRID, 1, BM), jnp.float32),
                   jax.ShapeDtypeStruct((N, H3), jnp.bfloat16),
                   jax.ShapeDtypeStruct((N, H3), jnp.bfloat16)),
    )(z, z, v1, v2)


def _mm_feedback(s, dv1, dv2, dvec, wl2):
    def kfn(s_ref, dv1_ref, dv2_ref, d_ref, wl2_ref, w2_ref):
        a1 = _dot(s_ref[...], dv1_ref[...])
        a2 = _dot(s_ref[...], dv2_ref[...])
        d = d_ref[0, 0, :]
        u = (jax.nn.relu(a1) + jax.nn.relu(a2)) * d[:, None]
        w2 = _dot(u.astype(jnp.bfloat16), wl2_ref[...].astype(jnp.bfloat16))
        w2_ref[...] = (w2 * d[:, None]).astype(jnp.bfloat16)

    return pl.pallas_call(
        kfn,
        grid=(GRID,),
        in_specs=[_rows((BM, N)), _full((N, H3)), _full((N, H3)),
                  _rows((1, 1, BM)), _full((H3, H2))],
        out_specs=_rows((BM, H2)),
        out_shape=jax.ShapeDtypeStruct((N, H2), jnp.bfloat16),
    )(s, dv1, dv2, dvec, wl2)


def _mm_update(s, w2, z, dvec):
    def kfn(s_ref, w2_ref, z_ref, d_ref, o_ref):
        d = d_ref[0, 0, :]
        upd = _dot(s_ref[...], w2_ref[...]) * d[:, None]
        o_ref[...] = (1.0 - AR) * z_ref[...] + AR * upd

    return pl.pallas_call(
        kfn,
        grid=(GRID,),
        in_specs=[_rows((BM, N)), _full((N, H2)), _rows((BM, H2)),
                  _rows((1, 1, BM))],
        out_specs=_rows((BM, H2)),
        out_shape=jax.ShapeDtypeStruct((N, H2), jnp.float32),
    )(s, w2, z, dvec)


def _mm_outer(upd):
    def kfn(ub_ref, uf_ref, o_ref):
        t = _dot_t(ub_ref[...], uf_ref[...])
        o_ref[...] = t.reshape(BM, N // SCW, SCW)

    return pl.pallas_call(
        kfn,
        grid=(GRID,),
        in_specs=[_rows((BM, H2)), _full((N, H2))],
        out_specs=_rows((BM, N // SCW, SCW)),
        out_shape=jax.ShapeDtypeStruct((N, N // SCW, SCW), jnp.float32),
    )(upd, upd)


def kernel(x, edge_index, W_enc0, W_mu, W_logstd, W_l0, W_l1, W_l2):
    src = edge_index[0].astype(jnp.int32)
    dst = edge_index[1].astype(jnp.int32)

    ones_blk = jnp.ones((CHUNK, SCW), jnp.float32)
    degp = _sc_gather_scatter(ones_blk, dst, dst, N, gather=False)

    h0st, v2 = _mm_xw(x, jnp.concatenate([W_enc0, W_l0], axis=1))
    h0pst = _mm_scale_h0(degp, h0st)

    src2 = jnp.concatenate([src, src + N])
    dst2 = jnp.concatenate([dst, dst + N])
    p1d = _sc_gather_scatter(h0pst.reshape(2 * N, SCW), src2, dst2, 2 * N)
    hz, hzp = _mm_hidden(degp, p1d.reshape(NC, 2, N, SCW), h0pst, W_mu)
    p2 = _sc_gather_scatter(hzp, src, dst, N)
    z, v1 = _mm_z(degp, p2, hzp, W_l1)

    s, dvec, dv1, dv2 = _mm_sig(z, v1, v2)
    w2 = _mm_feedback(s, dv1, dv2, dvec, W_l2)
    upd = _mm_update(s, w2, z, dvec)
    out = _mm_outer(upd)
    return out.reshape(-1)


# speedup vs baseline: 16.5209x; 1.0045x over previous
import functools

import jax
import jax.numpy as jnp
from jax import lax
from jax.experimental import pallas as pl
from jax.experimental.pallas import tpu as pltpu
from jax.experimental.pallas import tpu_sc as plsc

N = 4096
E = 131072
D = 512
H1 = 256
H2 = 128
H3 = 256
AR = 0.5

NC = 2
NS = 16
NW = NC * NS
CHUNK = 128
SCW = 128

BM = 512
GRID = N // BM

PREC = lax.Precision.DEFAULT


def _dot(a, b, prec=PREC):
    return lax.dot_general(a, b, (((1,), (0,)), ((), ())), precision=prec,
                           preferred_element_type=jnp.float32)


def _dot_t(a, b, prec=PREC):
    return lax.dot_general(a, b, (((1,), (1,)), ((), ())), precision=prec,
                           preferred_element_type=jnp.float32)


def _sc_gather_scatter(table, src, dst, an, gather=True):
    ne = src.shape[0]
    epw = ne // NW
    nch = epw // CHUNK
    rpt = an // NS
    src2d = src.reshape(NW, nch, CHUNK)
    dst2d = dst.reshape(NW, nch, CHUNK)
    mesh = plsc.VectorSubcoreMesh(core_axis_name="c", subcore_axis_name="s")

    @functools.partial(
        pl.kernel,
        mesh=mesh,
        out_type=jax.ShapeDtypeStruct((NC, an, SCW), jnp.float32),
        scratch_types=[
            pltpu.VMEM((nch, CHUNK), jnp.int32),
            pltpu.VMEM((nch, CHUNK), jnp.int32),
            pltpu.VMEM((2, CHUNK, SCW), jnp.float32),
            pltpu.VMEM_SHARED((an, SCW), jnp.float32),
            pltpu.SemaphoreType.DMA,
            pltpu.SemaphoreType.DMA,
        ],
    )
    def k(table_hbm, src_hbm, dst_hbm, zeros_hbm, out_hbm, sidx, didx, rows,
          acc, sem0, sem1):
        c = lax.axis_index("c")
        s = lax.axis_index("s")
        wid = s * NC + c
        pltpu.sync_copy(zeros_hbm, acc.at[pl.ds(s * rpt, rpt)])
        pltpu.sync_copy(dst_hbm.at[wid], didx)
        if gather:
            pltpu.sync_copy(src_hbm.at[wid], sidx)
        else:
            pltpu.sync_copy(table_hbm, rows.at[0])
        plsc.subcore_barrier()

        if gather:
            sems = (sem0, sem1)

            def gdesc(i, b):
                return pltpu.make_async_copy(
                    table_hbm.at[sidx.at[i]], rows.at[b], sems[b])

            for b in range(2):
                gdesc(b, b).start()

            def body(j, carry):
                for b in range(2):
                    i = j * 2 + b
                    gdesc(i, b).wait()
                    pltpu.sync_copy(rows.at[b], acc.at[didx.at[i]], add=True)

                    @pl.when(i + 2 < nch)
                    def _():
                        gdesc(i + 2, b).start()
                return carry

            lax.fori_loop(0, nch // 2, body, 0)
        else:
            def sdesc(i):
                return pltpu.make_async_copy(
                    rows.at[0], acc.at[didx.at[i]], sem0)

            def fire(i, carry):
                sdesc(i).start(add=True)
                return carry

            def drain(i, carry):
                sdesc(i).wait()
                return carry

            lax.fori_loop(0, nch, fire, 0)
            lax.fori_loop(0, nch, drain, 0)

        plsc.subcore_barrier()
        pltpu.sync_copy(acc.at[pl.ds(s * rpt, rpt)],
                        out_hbm.at[c, pl.ds(s * rpt, rpt)])

    return k(table, src2d, dst2d, jnp.zeros((rpt, SCW), jnp.float32))


def _full(shape):
    nd = len(shape)
    return pl.BlockSpec(shape, lambda i, _nd=nd: (0,) * _nd)


def _rows(shape_blk, axis=0):
    def imap(i):
        idx = [0] * len(shape_blk)
        idx[axis] = i
        return tuple(idx)
    return pl.BlockSpec(shape_blk, imap)


def _dinv_from_partials(degp_blk):
    deg = jnp.sum(degp_blk, axis=(0, 2)) * (1.0 / SCW) + 1.0
    return lax.rsqrt(deg)


def _mm_xw(x, wcat):
    def kfn(x_ref, w_ref, h0st_ref, v2_ref):
        t = _dot(x_ref[...], w_ref[...])
        h0st_ref[0] = t[:, :SCW]
        h0st_ref[1] = t[:, SCW:H1]
        v2_ref[...] = t[:, H1:]

    return pl.pallas_call(
        kfn,
        grid=(GRID,),
        in_specs=[_rows((BM, D)), _full((D, H1 + H3))],
        out_specs=(_rows((2, BM, SCW), axis=1), _rows((BM, H3))),
        out_shape=(jax.ShapeDtypeStruct((2, N, SCW), jnp.float32),
                   jax.ShapeDtypeStruct((N, H3), jnp.float32)),
    )(x, wcat)


def _mm_scale_h0(degp, h0st):
    def kfn(degp_ref, h0_ref, o_ref):
        dinv = _dinv_from_partials(degp_ref[...])
        o_ref[...] = h0_ref[...] * dinv[None, :, None]

    return pl.pallas_call(
        kfn,
        grid=(GRID,),
        in_specs=[_rows((NC, BM, SCW), axis=1), _rows((2, BM, SCW), axis=1)],
        out_specs=_rows((2, BM, SCW), axis=1),
        out_shape=jax.ShapeDtypeStruct((2, N, SCW), jnp.float32),
    )(degp, h0st)


def _mm_hidden(degp, p1d, h0pst, wmu):
    def kfn(degp_ref, p1_ref, h0p_ref, wmu_ref, hz_ref, hzp_ref):
        dinv = _dinv_from_partials(degp_ref[...])
        left = p1_ref[0, 0] + p1_ref[1, 0] + h0p_ref[0]
        right = p1_ref[0, 1] + p1_ref[1, 1] + h0p_ref[1]
        hidden1 = jax.nn.relu(
            jnp.concatenate([left, right], axis=1) * dinv[:, None])
        hz = _dot(hidden1, wmu_ref[...])
        hz_ref[...] = hz
        hzp_ref[...] = hz * dinv[:, None]

    return pl.pallas_call(
        kfn,
        grid=(GRID,),
        in_specs=[_rows((NC, BM, SCW), axis=1),
                  _rows((NC, 2, BM, SCW), axis=2),
                  _rows((2, BM, SCW), axis=1), _full((H1, H2))],
        out_specs=(_rows((BM, H2)), _rows((BM, H2))),
        out_shape=(jax.ShapeDtypeStruct((N, H2), jnp.float32),
                   jax.ShapeDtypeStruct((N, H2), jnp.float32)),
    )(degp, p1d, h0pst, wmu)


def _mm_z(degp, p2, hzp, wl1):
    def kfn(degp_ref, p2_ref, hzp_ref, wl1_ref, z_ref, zb_ref, v1_ref):
        dinv = _dinv_from_partials(degp_ref[...])
        z = (p2_ref[0] + p2_ref[1] + hzp_ref[...]) * dinv[:, None]
        z_ref[...] = z
        zb_ref[...] = z.astype(jnp.bfloat16)
        v1_ref[...] = _dot(z, wl1_ref[...])

    return pl.pallas_call(
        kfn,
        grid=(GRID,),
        in_specs=[_rows((NC, BM, SCW), axis=1), _rows((NC, BM, H2), axis=1),
                  _rows((BM, H2)), _full((H2, H3))],
        out_specs=(_rows((BM, H2)), _rows((BM, H2)), _rows((BM, H3))),
        out_shape=(jax.ShapeDtypeStruct((N, H2), jnp.float32),
                   jax.ShapeDtypeStruct((N, H2), jnp.bfloat16),
                   jax.ShapeDtypeStruct((N, H3), jnp.float32)),
    )(degp, p2, hzp, wl1)


def _mm_sig(zb16, v1, v2):
    def kfn(zb_ref, zf_ref, v1_ref, v2_ref, s_ref, d_ref, dv1_ref, dv2_ref):
        logits = _dot_t(zb_ref[...], zf_ref[...])
        sig = jax.nn.sigmoid(logits)
        s_ref[...] = sig.astype(jnp.bfloat16)
        d = lax.rsqrt(jnp.sum(sig, axis=1))
        d_ref[...] = d.reshape(1, 1, BM)
        dv1_ref[...] = (v1_ref[...] * d[:, None]).astype(jnp.bfloat16)
        dv2_ref[...] = (v2_ref[...] * d[:, None]).astype(jnp.bfloat16)

    return pl.pallas_call(
        kfn,
        grid=(GRID,),
        in_specs=[_rows((BM, H2)), _full((N, H2)), _rows((BM, H3)),
                  _rows((BM, H3))],
        out_specs=(_rows((BM, N)), _rows((1, 1, BM)), _rows((BM, H3)),
                   _rows((BM, H3))),
        out_shape=(jax.ShapeDtypeStruct((N, N), jnp.bfloat16),
                   jax.ShapeDtypeStruct((GRID, 1, BM), jnp.float32),
                   jax.ShapeDtypeStruct((N, H3), jnp.bfloat16),
                   jax.ShapeDtypeStruct((N, H3), jnp.bfloat16)),
    )(zb16, zb16, v1, v2)


def _mm_feedback(s, dv1, dv2, dvec, wl2):
    def kfn(s_ref, dv1_ref, dv2_ref, d_ref, wl2_ref, w2_ref):
        a1 = _dot(s_ref[...], dv1_ref[...])
        a2 = _dot(s_ref[...], dv2_ref[...])
        d = d_ref[0, 0, :]
        u = (jax.nn.relu(a1) + jax.nn.relu(a2)) * d[:, None]
        w2 = _dot(u.astype(jnp.bfloat16), wl2_ref[...].astype(jnp.bfloat16))
        w2_ref[...] = (w2 * d[:, None]).astype(jnp.bfloat16)

    return pl.pallas_call(
        kfn,
        grid=(GRID,),
        in_specs=[_rows((BM, N)), _full((N, H3)), _full((N, H3)),
                  _rows((1, 1, BM)), _full((H3, H2))],
        out_specs=_rows((BM, H2)),
        out_shape=jax.ShapeDtypeStruct((N, H2), jnp.bfloat16),
    )(s, dv1, dv2, dvec, wl2)


def _mm_update(s, w2, z, dvec):
    def kfn(s_ref, w2_ref, z_ref, d_ref, o_ref):
        d = d_ref[0, 0, :]
        upd = _dot(s_ref[...], w2_ref[...]) * d[:, None]
        o_ref[...] = ((1.0 - AR) * z_ref[...] + AR * upd).astype(jnp.bfloat16)

    return pl.pallas_call(
        kfn,
        grid=(GRID,),
        in_specs=[_rows((BM, N)), _full((N, H2)), _rows((BM, H2)),
                  _rows((1, 1, BM))],
        out_specs=_rows((BM, H2)),
        out_shape=jax.ShapeDtypeStruct((N, H2), jnp.bfloat16),
    )(s, w2, z, dvec)


def _mm_outer(upd):
    def kfn(ub_ref, uf_ref, o_ref):
        t = _dot_t(ub_ref[...], uf_ref[...])
        o_ref[...] = t.reshape(BM, N // SCW, SCW)

    return pl.pallas_call(
        kfn,
        grid=(GRID,),
        in_specs=[_rows((BM, H2)), _full((N, H2))],
        out_specs=_rows((BM, N // SCW, SCW)),
        out_shape=jax.ShapeDtypeStruct((N, N // SCW, SCW), jnp.float32),
    )(upd, upd)


def kernel(x, edge_index, W_enc0, W_mu, W_logstd, W_l0, W_l1, W_l2):
    src = edge_index[0].astype(jnp.int32)
    dst = edge_index[1].astype(jnp.int32)

    ones_blk = jnp.ones((CHUNK, SCW), jnp.float32)
    degp = _sc_gather_scatter(ones_blk, dst, dst, N, gather=False)

    h0st, v2 = _mm_xw(x, jnp.concatenate([W_enc0, W_l0], axis=1))
    h0pst = _mm_scale_h0(degp, h0st)

    src2 = jnp.concatenate([src, src + N])
    dst2 = jnp.concatenate([dst, dst + N])
    p1d = _sc_gather_scatter(h0pst.reshape(2 * N, SCW), src2, dst2, 2 * N)
    hz, hzp = _mm_hidden(degp, p1d.reshape(NC, 2, N, SCW), h0pst, W_mu)
    p2 = _sc_gather_scatter(hzp, src, dst, N)
    z, zb16, v1 = _mm_z(degp, p2, hzp, W_l1)

    s, dvec, dv1, dv2 = _mm_sig(zb16, v1, v2)
    w2 = _mm_feedback(s, dv1, dv2, dvec, W_l2)
    upd = _mm_update(s, w2, z, dvec)
    out = _mm_outer(upd)
    return out.reshape(-1)
